# baseline (speedup 1.0000x reference)
import functools
import jax
import jax.numpy as jnp
from jax import lax
from jax.experimental import pallas as pl
from jax.experimental.pallas import tpu as pltpu
from jax.experimental.pallas import tpu_sc as plsc

_CODE = 16
_CH = 2048
_NBUF = 2
_NW = 32
_GR = 128


def _enc_body(x_hbm, wet_ref, vq_hbm, xb, vb, in_sem, out_sem):
    n = x_hbm.shape[0] // _CH
    wet = wet_ref[...]

    def start_in(i, slot):
        pltpu.make_async_copy(
            x_hbm.at[pl.ds(i * _CH, _CH)], xb.at[slot], in_sem.at[slot]
        ).start()

    def wait_in(slot):
        pltpu.make_async_copy(
            x_hbm.at[pl.ds(0, _CH)], xb.at[slot], in_sem.at[slot]
        ).wait()

    def start_out(i, slot):
        pltpu.make_async_copy(
            vb.at[slot], vq_hbm.at[pl.ds(i * (_CH // 128), _CH // 128)],
            out_sem.at[slot]
        ).start()

    def wait_out(slot):
        pltpu.make_async_copy(
            vb.at[slot], vq_hbm.at[pl.ds(0, _CH // 128)], out_sem.at[slot]
        ).wait()

    for s in range(_NBUF):
        start_in(s, s)

    for i in range(n):
        slot = i % _NBUF
        wait_in(slot)
        x = xb[slot]
        h = lax.dot_general(x, wet, (((1,), (0,)), ((), ())),
                            preferred_element_type=jnp.float32)
        mx = jnp.max(h, axis=1, keepdims=True)
        iota = lax.broadcasted_iota(jnp.int32, h.shape, 1)
        first = jnp.min(jnp.where(h >= mx, iota, _CODE), axis=1, keepdims=True)
        if i + _NBUF < n:
            start_in(i + _NBUF, slot)
        if i >= _NBUF:
            wait_out(slot)
        vb[slot] = jnp.reshape(first, (_CH // 128, 128))
        start_out(i, slot)

    for i in range(max(n - _NBUF, 0), n):
        wait_out(i % _NBUF)


def _encode(x, wet):
    B, IN = x.shape
    return pl.pallas_call(
        _enc_body,
        in_specs=[
            pl.BlockSpec(memory_space=pl.ANY),
            pl.BlockSpec(memory_space=pltpu.VMEM),
        ],
        out_specs=pl.BlockSpec(memory_space=pl.ANY),
        out_shape=jax.ShapeDtypeStruct((B // 128, 128), jnp.int32),
        scratch_shapes=[
            pltpu.VMEM((_NBUF, _CH, IN), jnp.float32),
            pltpu.VMEM((_NBUF, _CH // 128, 128), jnp.int32),
            pltpu.SemaphoreType.DMA((_NBUF,)),
            pltpu.SemaphoreType.DMA((_NBUF,)),
        ],
        compiler_params=pltpu.CompilerParams(
            vmem_limit_bytes=100 * 1024 * 1024,
        ),
    )(x, wet)


def _decode(vq2d, wdt):
    B = vq2d.shape[0] * 128
    D = wdt.shape[1]
    rows_per_w = B // _NW
    rounds = rows_per_w // _GR
    mesh = plsc.VectorSubcoreMesh(core_axis_name="c", subcore_axis_name="s")

    @functools.partial(
        pl.kernel,
        mesh=mesh,
        out_type=jax.ShapeDtypeStruct((B, D), jnp.float32),
        scratch_types=[
            pltpu.VMEM((_GR,), jnp.int32),
            pltpu.VMEM((_GR, D), jnp.float32),
            pltpu.SemaphoreType.DMA,
        ],
    )
    def dec(vq_hbm, wdt_hbm, y_hbm, idx_v, rows_v, sem):
        wid = lax.axis_index("s") * 2 + lax.axis_index("c")
        base_row = wid * rounds

        def body(r, _):
            pltpu.sync_copy(vq_hbm.at[base_row + r], idx_v)
            pltpu.async_copy(wdt_hbm.at[idx_v], rows_v, sem).wait()
            pltpu.sync_copy(
                rows_v, y_hbm.at[pl.ds(wid * rows_per_w + r * _GR, _GR)]
            )
            return 0

        lax.fori_loop(0, rounds, body, 0)

    return dec(vq2d, wdt)


def kernel(x, W_enc, W_dec):
    vq2d = _encode(x, W_enc.T)
    return _decode(vq2d, W_dec.T)


# speedup vs baseline: 2.1945x; 2.1945x over previous
import jax
import jax.numpy as jnp
from jax import lax
from jax.experimental import pallas as pl
from jax.experimental.pallas import tpu as pltpu

_CODE = 16
_CH = 2048
_NBUF = 2


def _vq_body(x_hbm, wet_ref, wdt_ref, y_hbm, xb, yb, in_sem, out_sem):
    n = x_hbm.shape[0] // _CH
    wet = wet_ref[...]
    wdt = wdt_ref[...]

    def start_in(i, slot):
        pltpu.make_async_copy(
            x_hbm.at[pl.ds(i * _CH, _CH)], xb.at[slot], in_sem.at[slot]
        ).start()

    def wait_in(slot):
        pltpu.make_async_copy(
            x_hbm.at[pl.ds(0, _CH)], xb.at[slot], in_sem.at[slot]
        ).wait()

    def start_out(i, slot):
        pltpu.make_async_copy(
            yb.at[slot], y_hbm.at[pl.ds(i * _CH, _CH)], out_sem.at[slot]
        ).start()

    def wait_out(slot):
        pltpu.make_async_copy(
            yb.at[slot], y_hbm.at[pl.ds(0, _CH)], out_sem.at[slot]
        ).wait()

    for s in range(_NBUF):
        start_in(s, s)

    for i in range(n):
        slot = i % _NBUF
        wait_in(slot)
        x = xb[slot]
        h = lax.dot_general(x, wet, (((1,), (0,)), ((), ())),
                            preferred_element_type=jnp.float32)
        mx = jnp.max(h, axis=1, keepdims=True)
        iota = lax.broadcasted_iota(jnp.int32, h.shape, 1)
        first = jnp.min(jnp.where(h >= mx, iota, _CODE), axis=1, keepdims=True)
        onehot = (iota == first).astype(jnp.float32)
        if i + _NBUF < n:
            start_in(i + _NBUF, slot)
        if i >= _NBUF:
            wait_out(slot)
        yb[slot] = lax.dot_general(onehot, wdt, (((1,), (0,)), ((), ())),
                                   preferred_element_type=jnp.float32)
        start_out(i, slot)

    for i in range(max(n - _NBUF, 0), n):
        wait_out(i % _NBUF)


def kernel(x, W_enc, W_dec):
    B, IN = x.shape
    OUT = W_dec.shape[0]
    return pl.pallas_call(
        _vq_body,
        in_specs=[
            pl.BlockSpec(memory_space=pl.ANY),
            pl.BlockSpec(memory_space=pltpu.VMEM),
            pl.BlockSpec(memory_space=pltpu.VMEM),
        ],
        out_specs=pl.BlockSpec(memory_space=pl.ANY),
        out_shape=jax.ShapeDtypeStruct((B, OUT), jnp.float32),
        scratch_shapes=[
            pltpu.VMEM((_NBUF, _CH, IN), jnp.float32),
            pltpu.VMEM((_NBUF, _CH, OUT), jnp.float32),
            pltpu.SemaphoreType.DMA((_NBUF,)),
            pltpu.SemaphoreType.DMA((_NBUF,)),
        ],
        compiler_params=pltpu.CompilerParams(
            vmem_limit_bytes=100 * 1024 * 1024,
        ),
    )(x, W_enc.T, W_dec.T)
